# baseline (speedup 1.0000x reference)
import jax
import jax.numpy as jnp
from jax.experimental import pallas as pl
from jax.experimental.pallas import tpu as pltpu


def _affine_kernel(cam_ref, id_ref, img_ref, w_ref, b_ref, out_ref):
    bidx = pl.program_id(0)
    rem = id_ref[bidx] % 8
    sel = jax.lax.broadcasted_iota(jnp.int32, (8, 1), 0) == rem
    for c in range(3):
        wv = jnp.sum(jnp.where(sel, w_ref[0, :, c : c + 1], 0.0))
        bv = jnp.sum(jnp.where(sel, b_ref[0, :, c : c + 1], 0.0))
        out_ref[0, c] = img_ref[0, c] * wv + bv


def kernel(image, camindex, idindex, w, b):
    B, C, H, W = image.shape
    grid_spec = pltpu.PrefetchScalarGridSpec(
        num_scalar_prefetch=2,
        grid=(B,),
        in_specs=[
            pl.BlockSpec((1, C, H, W), lambda bi, cam, idx: (bi, 0, 0, 0)),
            pl.BlockSpec((1, 8, 3), lambda bi, cam, idx: (cam[bi], idx[bi] // 8, 0)),
            pl.BlockSpec((1, 8, 3), lambda bi, cam, idx: (cam[bi], idx[bi] // 8, 0)),
        ],
        out_specs=pl.BlockSpec((1, C, H, W), lambda bi, cam, idx: (bi, 0, 0, 0)),
    )
    return pl.pallas_call(
        _affine_kernel,
        grid_spec=grid_spec,
        out_shape=jax.ShapeDtypeStruct(image.shape, image.dtype),
    )(camindex, idindex, image, w, b)
